# speedup vs baseline: 3.1584x; 3.1584x over previous
import functools

import jax
import jax.numpy as jnp
from jax import lax
from jax.experimental import pallas as pl
from jax.experimental.pallas import tpu as pltpu
from jax.experimental.pallas import tpu_sc as plsc

_DIM = 128
_NC = 2
_NS = 16
_NW = _NC * _NS
_CHUNK = 128


@functools.lru_cache(maxsize=None)
def _make_gather(n_rows):
    rows_per_w = n_rows // _NW
    n_chunks = rows_per_w // _CHUNK
    idx_rows_per_w = rows_per_w // _CHUNK
    mesh = plsc.VectorSubcoreMesh(core_axis_name="c", subcore_axis_name="s")

    @functools.partial(
        pl.kernel,
        mesh=mesh,
        out_type=[
            jax.ShapeDtypeStruct((n_rows, _DIM), jnp.float32),
            jax.ShapeDtypeStruct((n_rows, _DIM), jnp.float32),
        ],
        scratch_types=[
            pltpu.VMEM((idx_rows_per_w, _CHUNK), jnp.int32),
            pltpu.VMEM((_CHUNK, _DIM), jnp.float32),
            pltpu.VMEM((_CHUNK, _DIM), jnp.float32),
            pltpu.SemaphoreType.DMA,
            pltpu.SemaphoreType.DMA,
        ],
    )
    def gather_kernel(cos_hbm, sin_hbm, idx_hbm, cos_out, sin_out,
                      idx_v, buf_a, buf_b, sem_a, sem_b):
        wid = lax.axis_index("s") * _NC + lax.axis_index("c")
        base = wid * rows_per_w
        pltpu.sync_copy(idx_hbm.at[pl.ds(wid * idx_rows_per_w, idx_rows_per_w)],
                        idx_v)

        steps = []
        for table, out in ((cos_hbm, cos_out), (sin_hbm, sin_out)):
            for c in range(n_chunks):
                steps.append((table, out, c))

        bufs = (buf_a, buf_b)
        sems = (sem_a, sem_b)
        handles = [None] * len(steps)

        def start(i):
            table, _, c = steps[i]
            handles[i] = pltpu.async_copy(
                table.at[idx_v.at[c]], bufs[i % 2], sems[i % 2])

        start(0)
        if len(steps) > 1:
            start(1)
        for i in range(len(steps)):
            _, out, c = steps[i]
            handles[i].wait()
            pltpu.sync_copy(bufs[i % 2],
                            out.at[pl.ds(base + c * _CHUNK, _CHUNK)])
            if i + 2 < len(steps):
                start(i + 2)

    return gather_kernel


def kernel(x, pos_ids, cos_cached, sin_cached):
    b, s = pos_ids.shape
    n_rows = b * s
    out_dt = x.dtype
    cos_t = cos_cached.astype(out_dt)
    sin_t = sin_cached.astype(out_dt)
    idx = pos_ids.reshape(n_rows // _CHUNK, _CHUNK).astype(jnp.int32)
    cos_r, sin_r = _make_gather(n_rows)(cos_t, sin_t, idx)
    return (cos_r.reshape(b, s, 1, _DIM),
            sin_r.reshape(b, s, 1, _DIM))


# speedup vs baseline: 3.1591x; 1.0002x over previous
import functools

import jax
import jax.numpy as jnp
from jax import lax
from jax.experimental import pallas as pl
from jax.experimental.pallas import tpu as pltpu
from jax.experimental.pallas import tpu_sc as plsc

_DIM = 128
_NC = 2
_NS = 16
_NW = _NC * _NS
_CHUNK = 128


@functools.lru_cache(maxsize=None)
def _make_gather(n_rows):
    rows_per_w = n_rows // _NW
    n_chunks = rows_per_w // _CHUNK
    idx_rows_per_w = rows_per_w // _CHUNK
    mesh = plsc.VectorSubcoreMesh(core_axis_name="c", subcore_axis_name="s")

    n_buf = 4

    @functools.partial(
        pl.kernel,
        mesh=mesh,
        out_type=[
            jax.ShapeDtypeStruct((n_rows, _DIM), jnp.float32),
            jax.ShapeDtypeStruct((n_rows, _DIM), jnp.float32),
        ],
        scratch_types=(
            [pltpu.VMEM((idx_rows_per_w, _CHUNK), jnp.int32)]
            + [pltpu.VMEM((_CHUNK, _DIM), jnp.float32)] * n_buf
            + [pltpu.SemaphoreType.DMA] * (2 * n_buf)
        ),
    )
    def gather_kernel(cos_hbm, sin_hbm, idx_hbm, cos_out, sin_out,
                      idx_v, *bufs_and_sems):
        bufs = bufs_and_sems[:n_buf]
        gsems = bufs_and_sems[n_buf:2 * n_buf]
        wsems = bufs_and_sems[2 * n_buf:]
        wid = lax.axis_index("s") * _NC + lax.axis_index("c")
        base = wid * rows_per_w
        pltpu.sync_copy(idx_hbm.at[pl.ds(wid * idx_rows_per_w, idx_rows_per_w)],
                        idx_v)

        steps = []
        for table, out in ((cos_hbm, cos_out), (sin_hbm, sin_out)):
            for c in range(n_chunks):
                steps.append((table, out, c))
        n = len(steps)

        ghandles = [None] * n
        whandles = [None] * n

        def start_gather(i):
            table, _, c = steps[i]
            ghandles[i] = pltpu.async_copy(
                table.at[idx_v.at[c]], bufs[i % n_buf], gsems[i % n_buf])

        for i in range(min(n_buf, n)):
            start_gather(i)
        for i in range(n):
            _, out, c = steps[i]
            ghandles[i].wait()
            whandles[i] = pltpu.async_copy(
                bufs[i % n_buf],
                out.at[pl.ds(base + c * _CHUNK, _CHUNK)],
                wsems[i % n_buf])
            if i + n_buf < n:
                whandles[i].wait()
                start_gather(i + n_buf)
        for i in range(max(0, n - n_buf), n):
            whandles[i].wait()

    return gather_kernel


def kernel(x, pos_ids, cos_cached, sin_cached):
    b, s = pos_ids.shape
    n_rows = b * s
    out_dt = x.dtype
    cos_t = cos_cached.astype(out_dt)
    sin_t = sin_cached.astype(out_dt)
    idx = pos_ids.reshape(n_rows // _CHUNK, _CHUNK).astype(jnp.int32)
    cos_r, sin_r = _make_gather(n_rows)(cos_t, sin_t, idx)
    return (cos_r.reshape(b, s, 1, _DIM),
            sin_r.reshape(b, s, 1, _DIM))


# speedup vs baseline: 3.2838x; 1.0395x over previous
import functools

import jax
import jax.numpy as jnp
from jax import lax
from jax.experimental import pallas as pl
from jax.experimental.pallas import tpu as pltpu
from jax.experimental.pallas import tpu_sc as plsc

_DIM = 128
_NC = 2
_NS = 16
_NW = _NC * _NS
_CHUNK = 128


@functools.lru_cache(maxsize=None)
def _make_gather(batch, seq):
    n_rows = batch * seq
    rows_per_w = n_rows // _NW
    n_chunks = rows_per_w // _CHUNK
    w_per_b = seq // rows_per_w
    mesh = plsc.VectorSubcoreMesh(core_axis_name="c", subcore_axis_name="s")
    n_buf = 4

    @functools.partial(
        pl.kernel,
        mesh=mesh,
        out_type=[
            jax.ShapeDtypeStruct((batch, seq, 1, _DIM), jnp.float32),
            jax.ShapeDtypeStruct((batch, seq, 1, _DIM), jnp.float32),
        ],
        scratch_types=(
            [pltpu.VMEM((rows_per_w,), jnp.int32)]
            + [pltpu.VMEM((_CHUNK, _DIM), jnp.float32)] * n_buf
            + [pltpu.SemaphoreType.DMA] * (2 * n_buf)
        ),
    )
    def gather_kernel(cos_hbm, sin_hbm, idx_hbm, cos_out, sin_out,
                      idx_v, *bufs_and_sems):
        bufs = bufs_and_sems[:n_buf]
        gsems = bufs_and_sems[n_buf:2 * n_buf]
        wsems = bufs_and_sems[2 * n_buf:]
        wid = lax.axis_index("s") * _NC + lax.axis_index("c")
        b = wid // w_per_b
        s0 = (wid % w_per_b) * rows_per_w
        pltpu.sync_copy(idx_hbm.at[b, pl.ds(s0, rows_per_w)], idx_v)

        steps = []
        for table, out in ((cos_hbm, cos_out), (sin_hbm, sin_out)):
            for c in range(n_chunks):
                steps.append((table, out, c))
        n = len(steps)

        ghandles = [None] * n
        whandles = [None] * n

        def start_gather(i):
            table, _, c = steps[i]
            ghandles[i] = pltpu.async_copy(
                table.at[idx_v.at[pl.ds(c * _CHUNK, _CHUNK)]],
                bufs[i % n_buf], gsems[i % n_buf])

        for i in range(min(n_buf, n)):
            start_gather(i)
        for i in range(n):
            _, out, c = steps[i]
            ghandles[i].wait()
            whandles[i] = pltpu.async_copy(
                bufs[i % n_buf],
                out.at[b, pl.ds(s0 + c * _CHUNK, _CHUNK), 0],
                wsems[i % n_buf])
            if i + n_buf < n:
                whandles[i].wait()
                start_gather(i + n_buf)
        for i in range(max(0, n - n_buf), n):
            whandles[i].wait()

    return gather_kernel


def kernel(x, pos_ids, cos_cached, sin_cached):
    b, s = pos_ids.shape
    out_dt = x.dtype
    cos_t = cos_cached.astype(out_dt)
    sin_t = sin_cached.astype(out_dt)
    idx = pos_ids.astype(jnp.int32)
    cos_r, sin_r = _make_gather(b, s)(cos_t, sin_t, idx)
    return (cos_r, sin_r)
